# baseline (speedup 1.0000x reference)
import functools

import jax
import jax.numpy as jnp
from jax import lax
from jax.experimental import pallas as pl
from jax.experimental.pallas import tpu as pltpu
from jax.experimental.pallas import tpu_sc as plsc

N = 10000
D = 64
NP = 10240
E = 640000
NW = 32
CH = 128
NCH = 164
EPT = NCH * CH
EPAD = NW * EPT
TROWS = NP // 16

_mesh = plsc.VectorSubcoreMesh(core_axis_name="c", subcore_axis_name="s")
_sc_params = pltpu.CompilerParams(use_tc_tiling_on_sc=False,
                                  needs_layout_passes=False)


@functools.partial(
    pl.kernel,
    out_type=jax.ShapeDtypeStruct((NW, NP), jnp.float32),
    mesh=_mesh,
    compiler_params=_sc_params,
    scratch_types=[
        pltpu.VMEM((NCH, CH), jnp.int32),
        pltpu.VMEM((NP,), jnp.float32),
        pltpu.VMEM((NP,), jnp.float32),
    ],
)
def _deg_kernel(dst_hbm, deg_out, dst_v, hist_v, hist_w):
    c = lax.axis_index("c")
    s = lax.axis_index("s")
    wid = s * 2 + c

    pltpu.sync_copy(dst_hbm.at[wid], dst_v)

    def _fill_z(i, _):
        hist_v[pl.ds(i * 16, 16)] = jnp.zeros((16,), jnp.float32)
        hist_w[pl.ds(i * 16, 16)] = jnp.zeros((16,), jnp.float32)
        return _
    lax.fori_loop(0, NP // 16, _fill_z, None)

    ones = jnp.full((16,), 1.0, jnp.float32)

    def _hist(r, _):
        for kk in range(4):
            for j in range(CH // 16):
                idx = dst_v[r * 4 + kk, pl.ds(j * 16, 16)]
                plsc.addupdate_scatter(hist_v if j % 2 == 0 else hist_w,
                                       [idx], ones)
        return _
    lax.fori_loop(0, NCH // 4, _hist, None)

    def _merge(i, _):
        sl = pl.ds(i * 16, 16)
        hist_v[sl] = hist_v[sl] + hist_w[sl]
        return _
    lax.fori_loop(0, NP // 16, _merge, None)

    pltpu.sync_copy(hist_v, deg_out.at[wid])


@functools.partial(
    pl.kernel,
    out_type=jax.ShapeDtypeStruct((2, NP, D), jnp.float32),
    mesh=_mesh,
    compiler_params=_sc_params,
    scratch_types=[
        pltpu.VMEM((NCH, CH), jnp.int32),
        pltpu.VMEM((NCH, CH), jnp.int32),
        [pltpu.VMEM((CH, D), jnp.float32) for _ in range(4)],
        [pltpu.SemaphoreType.DMA for _ in range(4)],
        [pltpu.SemaphoreType.DMA for _ in range(4)],
        pltpu.VMEM_SHARED((NP, D), jnp.float32),
    ],
)
def _agg_kernel(y_hbm, src_hbm, dst_hbm, p_out,
                src_v, dst_v, bufs, gsems, ssems, acc_sh):
    c = lax.axis_index("c")
    s = lax.axis_index("s")
    wid = s * 2 + c
    base = s * TROWS

    pltpu.sync_copy(src_hbm.at[wid], src_v)
    pltpu.sync_copy(dst_hbm.at[wid], dst_v)

    pltpu.sync_copy(y_hbm.at[pl.ds(base, TROWS)], acc_sh.at[pl.ds(base, TROWS)])
    plsc.subcore_barrier()

    NBUF = 4
    pltpu.async_copy(y_hbm.at[src_v.at[0]], bufs[0], gsems[0])
    pltpu.async_copy(y_hbm.at[src_v.at[1]], bufs[1], gsems[1])

    def _body(grp, _):
        base_ch = grp * NBUF
        for k in range(NBUF):
            cc = base_ch + k
            b = k
            bn = (k + 2) % NBUF
            pltpu.make_async_copy(y_hbm.at[src_v.at[cc]], bufs[b],
                                  gsems[b]).wait()
            pltpu.async_copy(bufs[b], acc_sh.at[dst_v.at[cc]], ssems[b],
                             add=True)
            @pl.when(cc >= 2)
            def _():
                pltpu.make_async_copy(
                    bufs[bn], acc_sh.at[dst_v.at[lax.max(cc - 2, 0)]],
                    ssems[bn]).wait()

            @pl.when(cc + 2 < NCH)
            def _():
                pltpu.async_copy(
                    y_hbm.at[src_v.at[lax.min(cc + 2, NCH - 1)]],
                    bufs[bn], gsems[bn])
        return _
    lax.fori_loop(0, NCH // NBUF, _body, None)

    pltpu.make_async_copy(bufs[2], acc_sh.at[dst_v.at[NCH - 2]],
                          ssems[2]).wait()
    pltpu.make_async_copy(bufs[3], acc_sh.at[dst_v.at[NCH - 1]],
                          ssems[3]).wait()

    plsc.subcore_barrier()
    pltpu.sync_copy(acc_sh.at[pl.ds(base, TROWS)], p_out.at[c, pl.ds(base, TROWS)])


def _y_body(dis_ref, x_ref, y_ref):
    y_ref[...] = dis_ref[...] * x_ref[...]


def _mm_body(dis_ref, p_ref, y_ref, w_ref, b_ref, out_ref, h_ref):
    i = pl.program_id(0)

    @pl.when(i == 0)
    def _():
        q = p_ref[0] + p_ref[1] - y_ref[...]
        pre = dis_ref[...] * q
        hh = (jnp.dot(pre, w_ref[...], preferred_element_type=jnp.float32)
              + b_ref[...])
        h_ref[...] = jnp.maximum(hh, 0.0).astype(jnp.bfloat16)

    hi = h_ref[pl.ds(i * BM, BM), :]
    out_ref[...] = lax.dot_general(
        hi, h_ref[pl.ds(0, N), :], (((1,), (1,)), ((), ())),
        preferred_element_type=jnp.float32)


BM = 400


def kernel(x, edge_index, W, b):
    ei = edge_index.astype(jnp.int32)
    src = jnp.concatenate(
        [ei[0], jnp.arange(EPAD - E, dtype=jnp.int32) % NP])
    dst = jnp.concatenate([ei[1], jnp.full((EPAD - E,), N, jnp.int32)])
    src3 = src.reshape(NW, NCH, CH)
    dst3 = dst.reshape(NW, NCH, CH)
    xp = jnp.pad(x, ((0, NP - N), (0, 0)))

    deg_p = _deg_kernel(dst3)
    dis = lax.rsqrt(jnp.sum(deg_p, axis=0) + 1.0)[:, None]

    y = pl.pallas_call(
        _y_body,
        out_shape=jax.ShapeDtypeStruct((NP, D), jnp.float32),
    )(dis, xp)

    p = _agg_kernel(y, src3, dst3)

    adj = pl.pallas_call(
        _mm_body,
        grid=(N // BM,),
        in_specs=[
            pl.BlockSpec((NP, 1), lambda i: (0, 0)),
            pl.BlockSpec((2, NP, D), lambda i: (0, 0, 0)),
            pl.BlockSpec((NP, D), lambda i: (0, 0)),
            pl.BlockSpec((D, D), lambda i: (0, 0)),
            pl.BlockSpec((1, D), lambda i: (0, 0)),
        ],
        out_specs=pl.BlockSpec((BM, N), lambda i: (i, 0)),
        out_shape=jax.ShapeDtypeStruct((N, N), jnp.float32),
        scratch_shapes=[pltpu.VMEM((NP, D), jnp.bfloat16)],
    )(dis, p, y, W, b[None, :])
    return adj


# speedup vs baseline: 1.0369x; 1.0369x over previous
import functools

import jax
import jax.numpy as jnp
from jax import lax
from jax.experimental import pallas as pl
from jax.experimental.pallas import tpu as pltpu
from jax.experimental.pallas import tpu_sc as plsc

N = 10000
D = 64
NP = 10240
E = 640000
NW = 32
CH = 128
NCH = 160
EPT = NCH * CH
EPAD = NW * EPT
TROWS = NP // 16

_mesh = plsc.VectorSubcoreMesh(core_axis_name="c", subcore_axis_name="s")
_sc_params = pltpu.CompilerParams(use_tc_tiling_on_sc=False,
                                  needs_layout_passes=False)


@functools.partial(
    pl.kernel,
    out_type=jax.ShapeDtypeStruct((NW, NP), jnp.float32),
    mesh=_mesh,
    compiler_params=_sc_params,
    scratch_types=[
        pltpu.VMEM((NCH, CH), jnp.int32),
        pltpu.VMEM((NP,), jnp.float32),
    ],
)
def _deg_kernel(dst_hbm, deg_out, dst_v, hist_v):
    c = lax.axis_index("c")
    s = lax.axis_index("s")
    wid = s * 2 + c

    pltpu.sync_copy(dst_hbm.at[wid], dst_v)

    def _fill_z(i, _):
        hist_v[pl.ds(i * 16, 16)] = jnp.zeros((16,), jnp.float32)
        return _
    lax.fori_loop(0, NP // 16, _fill_z, None)

    ones = jnp.full((16,), 1.0, jnp.float32)

    def _hist(r, _):
        for kk in range(4):
            for j in range(CH // 16):
                idx = dst_v[r * 4 + kk, pl.ds(j * 16, 16)]
                plsc.addupdate_scatter(hist_v, [idx], ones)
        return _
    lax.fori_loop(0, NCH // 4, _hist, None)

    pltpu.sync_copy(hist_v, deg_out.at[wid])


@functools.partial(
    pl.kernel,
    out_type=jax.ShapeDtypeStruct((2, NP, D), jnp.float32),
    mesh=_mesh,
    compiler_params=_sc_params,
    scratch_types=[
        pltpu.VMEM((NCH, CH), jnp.int32),
        pltpu.VMEM((NCH, CH), jnp.int32),
        [pltpu.VMEM((CH, D), jnp.float32) for _ in range(4)],
        [pltpu.SemaphoreType.DMA for _ in range(4)],
        [pltpu.SemaphoreType.DMA for _ in range(4)],
        pltpu.VMEM_SHARED((NP, D), jnp.float32),
    ],
)
def _agg_kernel(y_hbm, src_hbm, dst_hbm, p_out,
                src_v, dst_v, bufs, gsems, ssems, acc_sh):
    c = lax.axis_index("c")
    s = lax.axis_index("s")
    wid = s * 2 + c
    base = s * TROWS

    pltpu.sync_copy(src_hbm.at[wid], src_v)
    pltpu.sync_copy(dst_hbm.at[wid], dst_v)

    pltpu.sync_copy(y_hbm.at[pl.ds(base, TROWS)], acc_sh.at[pl.ds(base, TROWS)])
    plsc.subcore_barrier()

    NBUF = 4
    pltpu.async_copy(y_hbm.at[src_v.at[0]], bufs[0], gsems[0])
    pltpu.async_copy(y_hbm.at[src_v.at[1]], bufs[1], gsems[1])

    def _body(grp, _):
        base_ch = grp * NBUF
        for k in range(NBUF):
            cc = base_ch + k
            b = k
            bn = (k + 2) % NBUF
            pltpu.make_async_copy(y_hbm.at[src_v.at[cc]], bufs[b],
                                  gsems[b]).wait()
            pltpu.async_copy(bufs[b], acc_sh.at[dst_v.at[cc]], ssems[b],
                             add=True)
            @pl.when(cc >= 2)
            def _():
                pltpu.make_async_copy(
                    bufs[bn], acc_sh.at[dst_v.at[lax.max(cc - 2, 0)]],
                    ssems[bn]).wait()

            @pl.when(cc + 2 < NCH)
            def _():
                pltpu.async_copy(
                    y_hbm.at[src_v.at[lax.min(cc + 2, NCH - 1)]],
                    bufs[bn], gsems[bn])
        return _
    lax.fori_loop(0, NCH // NBUF, _body, None)

    pltpu.make_async_copy(bufs[2], acc_sh.at[dst_v.at[NCH - 2]],
                          ssems[2]).wait()
    pltpu.make_async_copy(bufs[3], acc_sh.at[dst_v.at[NCH - 1]],
                          ssems[3]).wait()

    plsc.subcore_barrier()
    pltpu.sync_copy(acc_sh.at[pl.ds(base, TROWS)], p_out.at[c, pl.ds(base, TROWS)])


def _y_body(dis_ref, x_ref, y_ref):
    y_ref[...] = dis_ref[...] * x_ref[...]


def _mm_body(dis_ref, p_ref, y_ref, w_ref, b_ref, out_ref, h_ref):
    i = pl.program_id(0)

    @pl.when(i == 0)
    def _():
        q = p_ref[0] + p_ref[1] - y_ref[...]
        pre = dis_ref[...] * q
        hh = (jnp.dot(pre, w_ref[...], preferred_element_type=jnp.float32)
              + b_ref[...])
        h_ref[...] = jnp.maximum(hh, 0.0)

    hi = h_ref[pl.ds(i * BM, BM), :]
    out_ref[...] = lax.dot_general(
        hi, h_ref[pl.ds(0, N), :], (((1,), (1,)), ((), ())),
        preferred_element_type=jnp.float32)


BM = 400


def kernel(x, edge_index, W, b):
    ei = edge_index.astype(jnp.int32)
    src = jnp.concatenate(
        [ei[0], jnp.arange(EPAD - E, dtype=jnp.int32) % NP])
    dst = jnp.concatenate([ei[1], jnp.full((EPAD - E,), N, jnp.int32)])
    src3 = src.reshape(NW, NCH, CH)
    dst3 = dst.reshape(NW, NCH, CH)
    xp = jnp.pad(x, ((0, NP - N), (0, 0)))

    deg_p = _deg_kernel(dst3)
    dis = lax.rsqrt(jnp.sum(deg_p, axis=0) + 1.0)[:, None]

    y = pl.pallas_call(
        _y_body,
        out_shape=jax.ShapeDtypeStruct((NP, D), jnp.float32),
    )(dis, xp)

    p = _agg_kernel(y, src3, dst3)

    adj = pl.pallas_call(
        _mm_body,
        grid=(N // BM,),
        in_specs=[
            pl.BlockSpec((NP, 1), lambda i: (0, 0)),
            pl.BlockSpec((2, NP, D), lambda i: (0, 0, 0)),
            pl.BlockSpec((NP, D), lambda i: (0, 0)),
            pl.BlockSpec((D, D), lambda i: (0, 0)),
            pl.BlockSpec((1, D), lambda i: (0, 0)),
        ],
        out_specs=pl.BlockSpec((BM, N), lambda i: (i, 0)),
        out_shape=jax.ShapeDtypeStruct((N, N), jnp.float32),
        scratch_shapes=[pltpu.VMEM((NP, D), jnp.float32)],
    )(dis, p, y, W, b[None, :])
    return adj
